# Initial kernel scaffold; baseline (speedup 1.0000x reference)
import jax
import jax.numpy as jnp
from jax.experimental import pallas as pl


def kernel(x, weight, codebook):
    raise NotImplementedError("write your pallas kernel here")


# speedup vs baseline: 16.8440x; 16.8440x over previous
import jax
import jax.numpy as jnp
from jax.experimental import pallas as pl
from jax.experimental.pallas import tpu as pltpu

O_CH, I_CH, KS = 192, 192, 3
NW = O_CH * I_CH * KS * KS
NEMB = 256
WROWS = NW // 128
BW = 288
GVQ = WROWS // BW
COMMIT = 0.25
MAXV = 255.0
SP = 30
OS = 28
B = 4


def _sort_body(cb_row_ref, cb_col_ref, s_ref, m_ref, d_ref):
    row = cb_row_ref[...]
    col = cb_col_ref[...]
    ii = jax.lax.broadcasted_iota(jnp.int32, (NEMB, NEMB), 0)
    jj = jax.lax.broadcasted_iota(jnp.int32, (NEMB, NEMB), 1)
    less = (row < col) | ((row == col) & (jj < ii))
    rank = jnp.sum(less.astype(jnp.int32), axis=1, keepdims=True)
    s = jnp.sum(jnp.where(rank == jj, col, 0.0), axis=0, keepdims=True)
    s_next = jnp.sum(jnp.where(rank == jj + 1, col, 0.0), axis=0, keepdims=True)
    lane = jax.lax.broadcasted_iota(jnp.int32, (1, NEMB), 1)
    inf = jnp.float32(jnp.inf)
    s_ref[...] = s
    m_ref[...] = jnp.where(lane == NEMB - 1, inf, (s + s_next) * 0.5)
    d_ref[...] = jnp.where(lane == NEMB - 1, 0.0, s_next - s)


def _vq_body(w_ref, s_ref, m_ref, d_ref, q_ref, red_ref):
    g = pl.program_id(0)
    w = w_ref[...]
    s0 = s_ref[0, 0]

    def step(k, q):
        mk = m_ref[0, k]
        dk = d_ref[0, k]
        return q + jnp.where(w > mk, dk, 0.0)

    q = jax.lax.fori_loop(0, NEMB - 1, step, jnp.full_like(w, s0), unroll=8)
    q_ref[...] = q
    e = q - w
    esum = jnp.sum(e * e)
    dsum = jnp.sum(w * w + q * q - 2.0 * w * q)
    qmax = jnp.max(jnp.abs(q))
    lane = jax.lax.broadcasted_iota(jnp.int32, (1, 128), 1)
    contrib = jnp.where(lane == 0, esum,
                        jnp.where(lane == 1, dsum,
                                  jnp.where(lane == 2, qmax, 0.0)))

    @pl.when(g == 0)
    def _():
        red_ref[...] = jnp.zeros((1, 128), jnp.float32)

    prev = red_ref[...]
    red_ref[...] = jnp.where(lane < 2, prev + contrib,
                             jnp.maximum(prev, contrib))


def _conv_body(x_ref, w9_ref, red_ref, out_ref, loss_ref):
    b = pl.program_id(0)
    esum = red_ref[0, 0]
    dsum = red_ref[0, 1]
    qmax = red_ref[0, 2]
    max_abs = jnp.where(qmax > 0.0, qmax, 1.0)

    w9 = w9_ref[...]
    wpos = jnp.maximum(w9, 0.0)
    wneg = jnp.maximum(-w9, 0.0)
    wint = jnp.round(wpos / max_abs * MAXV) - jnp.round(wneg / max_abs * MAXV)
    wb = wint.astype(jnp.bfloat16)

    x = x_ref[0]
    xq = jnp.round(jnp.clip(x, -8.0, 8.0 - 0.0625) * 16.0)
    xb = xq.astype(jnp.bfloat16)

    acc = jnp.zeros((OS * OS, O_CH), jnp.float32)
    for t in range(KS * KS):
        dy, dx = t // KS, t % KS
        p = jax.lax.dot(xb, wb[t], preferred_element_type=jnp.float32)
        pw = p.reshape(SP, SP, O_CH)[dy:dy + OS, dx:dx + OS, :]
        acc = acc + pw.reshape(OS * OS, O_CH)
    out_ref[0] = acc * (max_abs / (MAXV * 16.0))

    @pl.when(b == 0)
    def _():
        e_l = esum / NW
        avg = dsum / NW
        scale = jnp.where(avg < 0.001, 0.1, jnp.where(avg < 0.01, 0.5, 1.0))
        loss = e_l + COMMIT * scale * e_l
        loss_ref[...] = jnp.full((1, 128), loss)


def kernel(x, weight, codebook):
    cb_row = codebook.reshape(1, NEMB)
    cb_col = codebook.reshape(NEMB, 1)
    s, m, d = pl.pallas_call(
        _sort_body,
        out_shape=[jax.ShapeDtypeStruct((1, NEMB), jnp.float32)] * 3,
    )(cb_row, cb_col)

    w_flat = weight.reshape(WROWS, 128)
    smem = pl.BlockSpec(memory_space=pltpu.SMEM)
    q_flat, red = pl.pallas_call(
        _vq_body,
        grid=(GVQ,),
        in_specs=[pl.BlockSpec((BW, 128), lambda g: (g, 0)), smem, smem, smem],
        out_specs=[pl.BlockSpec((BW, 128), lambda g: (g, 0)),
                   pl.BlockSpec((1, 128), lambda g: (0, 0))],
        out_shape=[jax.ShapeDtypeStruct((WROWS, 128), jnp.float32),
                   jax.ShapeDtypeStruct((1, 128), jnp.float32)],
    )(w_flat, s, m, d)

    w9 = q_flat.reshape(O_CH, I_CH, KS * KS).transpose(2, 1, 0)
    xpad = jnp.pad(x, ((0, 0), (0, 0), (1, 1), (1, 1)))
    xt = xpad.transpose(0, 2, 3, 1).reshape(B, SP * SP, I_CH)

    out_t, loss_arr = pl.pallas_call(
        _conv_body,
        grid=(B,),
        in_specs=[pl.BlockSpec((1, SP * SP, I_CH), lambda b: (b, 0, 0)),
                  pl.BlockSpec((KS * KS, I_CH, O_CH), lambda b: (0, 0, 0)),
                  smem],
        out_specs=[pl.BlockSpec((1, OS * OS, O_CH), lambda b: (b, 0, 0)),
                   pl.BlockSpec((1, 128), lambda b: (0, 0))],
        out_shape=[jax.ShapeDtypeStruct((B, OS * OS, O_CH), jnp.float32),
                   jax.ShapeDtypeStruct((1, 128), jnp.float32)],
    )(xt, w9, red)

    out = out_t.transpose(0, 2, 1).reshape(B, O_CH, OS, OS)
    return out, loss_arr[0, 0]
